# baseline (speedup 1.0000x reference)
import jax
import jax.numpy as jnp
from jax import lax
from jax.experimental import pallas as pl
from jax.experimental.pallas import tpu as pltpu
from jax.experimental.pallas import tpu_sc as plsc

NUM_NODES = 1000000
NODE_DIM = 64
HIDDEN = 128
B = 16384

NC = 2
NS = 16
NW = NC * NS
B_PER_W = B // NW
CHUNK = 128
NCHUNK = B_PER_W // CHUNK
NBUF = 4
VROWS = NUM_NODES // 2


def _gather_body(ws_hbm, d_hbm, src_hbm, dst_hbm, out_hbm,
                 src_i, dst_i, bufs, sems):
    wid = lax.axis_index("s") * NC + lax.axis_index("c")
    base = wid * B_PER_W
    pltpu.sync_copy(src_hbm.at[pl.ds(wid * NCHUNK, NCHUNK)], src_i)
    pltpu.sync_copy(dst_hbm.at[pl.ds(wid * NCHUNK, NCHUNK)], dst_i)

    tasks = []
    for j in range(NCHUNK):
        tasks.append((ws_hbm, src_i, j, 0))
        tasks.append((ws_hbm, dst_i, j, 1))
        tasks.append((d_hbm, src_i, j, 2))
        tasks.append((d_hbm, dst_i, j, 3))

    copies = [None] * NBUF
    pending = [None] * NBUF
    for t, (table, idxr, j, g) in enumerate(tasks):
        b = t % NBUF
        if copies[b] is not None:
            copies[b].wait()
            pg, pj = pending[b]
            pltpu.sync_copy(bufs[b], out_hbm.at[pg, pl.ds(base + pj * CHUNK, CHUNK)])
        copies[b] = pltpu.async_copy(table.at[idxr.at[j]], bufs[b], sems[b])
        pending[b] = (g, j)
    for b in range(NBUF):
        copies[b].wait()
        pg, pj = pending[b]
        pltpu.sync_copy(bufs[b], out_hbm.at[pg, pl.ds(base + pj * CHUNK, CHUNK)])


def _sc_gather(Wv, Dv, src_g, dst_g):
    mesh = plsc.VectorSubcoreMesh(core_axis_name="c", subcore_axis_name="s",
                                  num_cores=NC, num_subcores=NS)
    scratch = (
        [pltpu.VMEM((NCHUNK, CHUNK), jnp.int32)] * 2
        + [pltpu.VMEM((CHUNK, 2 * NODE_DIM), jnp.float32)] * NBUF
        + [pltpu.SemaphoreType.DMA] * NBUF
    )

    def body(ws_hbm, d_hbm, src_hbm, dst_hbm, out_hbm, *rest):
        src_i, dst_i = rest[0], rest[1]
        bufs = list(rest[2:2 + NBUF])
        sems = list(rest[2 + NBUF:])
        _gather_body(ws_hbm, d_hbm, src_hbm, dst_hbm, out_hbm,
                     src_i, dst_i, bufs, sems)

    k = pl.kernel(
        body,
        out_type=jax.ShapeDtypeStruct((4, B, 2 * NODE_DIM), jnp.float32),
        mesh=mesh,
        scratch_types=scratch,
    )
    return k(Wv, Dv, src_g, dst_g)


def _mlp_body(hp_ref, s_ref, w1t_ref, b1_ref, w2t_ref, b2_ref, out_ref):
    acc = None
    for g in range(4):
        row = hp_ref[g]
        s = s_ref[:, g:g + 1]
        lo = row[:, 0:NODE_DIM]
        hi = row[:, NODE_DIM:2 * NODE_DIM]
        x = lo + s * (hi - lo)
        part = jnp.dot(x, w1t_ref[g * NODE_DIM:(g + 1) * NODE_DIM, :],
                       preferred_element_type=jnp.float32)
        acc = part if acc is None else acc + part
    h1 = jnp.maximum(acc + b1_ref[...], 0.0)
    out_ref[...] = jnp.dot(h1, w2t_ref[...],
                           preferred_element_type=jnp.float32) + b2_ref[...]


def _tc_mlp(hp, svec, W1, b1, W2, b2):
    blk = 2048
    grid = (B // blk,)
    w1t = W1.T
    w2t = W2.T
    out = pl.pallas_call(
        _mlp_body,
        grid=grid,
        in_specs=[
            pl.BlockSpec((4, blk, 2 * NODE_DIM), lambda i: (0, i, 0)),
            pl.BlockSpec((blk, 4), lambda i: (i, 0)),
            pl.BlockSpec((4 * NODE_DIM, HIDDEN), lambda i: (0, 0)),
            pl.BlockSpec((1, HIDDEN), lambda i: (0, 0)),
            pl.BlockSpec((HIDDEN, 1), lambda i: (0, 0)),
            pl.BlockSpec((1, 1), lambda i: (0, 0)),
        ],
        out_specs=pl.BlockSpec((blk, 1), lambda i: (i, 0)),
        out_shape=jax.ShapeDtypeStruct((B, 1), jnp.float32),
    )(hp, svec, w1t, b1.reshape(1, HIDDEN), w2t, b2.reshape(1, 1))
    return out.reshape(B)


def kernel(src, dst, ts, W_static, D, W1, b1, W2, b2):
    src32 = src.astype(jnp.int32)
    dst32 = dst.astype(jnp.int32)
    Wv = W_static.reshape(VROWS, 2 * NODE_DIM)
    Dv = D.reshape(VROWS, 2 * NODE_DIM)
    src_g = (src32 // 2).reshape(NW * NCHUNK, CHUNK)
    dst_g = (dst32 // 2).reshape(NW * NCHUNK, CHUNK)
    ss = (src32 % 2).astype(jnp.float32)
    sd = (dst32 % 2).astype(jnp.float32)
    svec = jnp.stack([ss, sd, ss, sd], axis=1)
    hp = _sc_gather(Wv, Dv, src_g, dst_g)
    return _tc_mlp(hp, svec, W1, b1, W2, b2)


# speedup vs baseline: 1.4231x; 1.4231x over previous
import jax
import jax.numpy as jnp
from jax import lax
from jax.experimental import pallas as pl
from jax.experimental.pallas import tpu as pltpu
from jax.experimental.pallas import tpu_sc as plsc

NUM_NODES = 1000000
NODE_DIM = 64
HIDDEN = 128
B = 16384

NC = 2
NS = 16
NW = NC * NS
B_PER_W = B // NW
L = 16
NTILE = NUM_NODES // 8
CHUNK = 128
NCHUNK = B_PER_W // CHUNK
VEC_PER_CHUNK = CHUNK // L


def _sc_gather_one(T, src, dst):
    mesh = plsc.VectorSubcoreMesh(core_axis_name="c", subcore_axis_name="s",
                                  num_cores=NC, num_subcores=NS)
    scratch = [
        pltpu.VMEM((B_PER_W,), jnp.int32),
        pltpu.VMEM((B_PER_W,), jnp.int32),
        pltpu.VMEM((L, 8, NODE_DIM), jnp.float32),
        pltpu.VMEM((L, 8, NODE_DIM), jnp.float32),
        pltpu.VMEM((CHUNK, 2 * NODE_DIM), jnp.float32),
        pltpu.SemaphoreType.DMA,
        pltpu.SemaphoreType.DMA,
    ]

    def body(t_hbm, src_hbm, dst_hbm, out_hbm,
             src_i, dst_i, ga, gb, pbuf, sem_a, sem_b):
        wid = lax.axis_index("s") * NC + lax.axis_index("c")
        base = wid * B_PER_W
        pltpu.sync_copy(src_hbm.at[pl.ds(base, B_PER_W)], src_i)
        pltpu.sync_copy(dst_hbm.at[pl.ds(base, B_PER_W)], dst_i)

        for g, idx_v in ((0, src_i), (1, dst_i)):
            tv = t_hbm.reshape(NTILE, 8, NODE_DIM)

            def issue_vec(off, grp, sem, idx_v=idx_v, tv=tv):
                vec = idx_v[pl.ds(off, L)]
                for j in range(L):
                    t = vec[j] // 8
                    pltpu.async_copy(tv.at[t], grp.at[j], sem)
                return vec

            def drain(grp, sem, tv=tv):
                pltpu.make_async_copy(tv.at[pl.ds(0, L)], grp, sem).wait()

            def extract(vec, grp, row0):
                for j in range(L):
                    s = vec[j] % 8
                    for c in range(NODE_DIM // L):
                        pbuf[row0 + j, pl.ds(c * L, L)] = \
                            grp[j, s, pl.ds(c * L, L)]

            def chunk_body(ch, _, idx_v=idx_v, tv=tv, g=g):
                def pair(k, _):
                    off = ch * CHUNK + 2 * k * L
                    vec_a = issue_vec(off, ga, sem_a)
                    vec_b = issue_vec(off + L, gb, sem_b)
                    drain(ga, sem_a)
                    extract(vec_a, ga, 2 * k * L)
                    drain(gb, sem_b)
                    extract(vec_b, gb, 2 * k * L + L)
                    return ()

                lax.fori_loop(0, VEC_PER_CHUNK // 2, pair, ())
                pltpu.sync_copy(
                    pbuf, out_hbm.at[g, pl.ds(base + ch * CHUNK, CHUNK)])
                return ()

            lax.fori_loop(0, NCHUNK, chunk_body, ())

    k = pl.kernel(
        body,
        out_type=jax.ShapeDtypeStruct((2, B, 2 * NODE_DIM), jnp.float32),
        mesh=mesh,
        scratch_types=scratch,
    )
    return k(T, src, dst)


def _mlp_body(hs_ref, hd_ref, w1t_ref, b1_ref, w2t_ref, b2_ref, out_ref):
    acc = None
    for g in range(4):
        ref = (hs_ref, hd_ref)[g // 2]
        part = jnp.dot(ref[g % 2][:, 0:NODE_DIM],
                       w1t_ref[g * NODE_DIM:(g + 1) * NODE_DIM, :],
                       preferred_element_type=jnp.float32)
        acc = part if acc is None else acc + part
    h1 = jnp.maximum(acc + b1_ref[...], 0.0)
    out_ref[...] = jnp.dot(h1, w2t_ref[...],
                           preferred_element_type=jnp.float32) + b2_ref[...]


def _tc_mlp(hs, hd, W1, b1, W2, b2):
    blk = 2048
    grid = (B // blk,)
    w1t = W1.T
    w2t = W2.T
    out = pl.pallas_call(
        _mlp_body,
        grid=grid,
        in_specs=[
            pl.BlockSpec((2, blk, 2 * NODE_DIM), lambda i: (0, i, 0)),
            pl.BlockSpec((2, blk, 2 * NODE_DIM), lambda i: (0, i, 0)),
            pl.BlockSpec((4 * NODE_DIM, HIDDEN), lambda i: (0, 0)),
            pl.BlockSpec((1, HIDDEN), lambda i: (0, 0)),
            pl.BlockSpec((HIDDEN, 1), lambda i: (0, 0)),
            pl.BlockSpec((1, 1), lambda i: (0, 0)),
        ],
        out_specs=pl.BlockSpec((blk, 1), lambda i: (i, 0)),
        out_shape=jax.ShapeDtypeStruct((B, 1), jnp.float32),
    )(hs, hd, w1t, b1.reshape(1, HIDDEN), w2t, b2.reshape(1, 1))
    return out.reshape(B)


def kernel(src, dst, ts, W_static, D, W1, b1, W2, b2):
    src32 = src.astype(jnp.int32)
    dst32 = dst.astype(jnp.int32)
    hs = _sc_gather_one(W_static, src32, dst32)
    hd = _sc_gather_one(D, src32, dst32)
    return _tc_mlp(hs, hd, W1, b1, W2, b2)
